# Initial kernel scaffold; baseline (speedup 1.0000x reference)
import jax
import jax.numpy as jnp
from jax.experimental import pallas as pl


def kernel(sim_mat):
    raise NotImplementedError("write your pallas kernel here")


# speedup vs baseline: 8.0930x; 8.0930x over previous
import jax
import jax.numpy as jnp
from jax.experimental import pallas as pl
from jax.experimental.pallas import tpu as pltpu

_B = 4096
_BR = 256
_NSTEPS = _B // _BR
_MARGIN = 0.2
_NEG = -1e30


def _gumbel(bits):
    u = (bits >> 8).astype(jnp.float32) * (1.0 / 16777216.0) + (0.5 / 16777216.0)
    return -jnp.log(-jnp.log(u))


def _body(s_ref, out_ref, colk, colp, diag, acc):
    step = pl.program_id(0)
    s = s_ref[...]
    u = jnp.maximum(2.0 - 2.0 * s, 0.25)
    lw = -255.0 * jnp.log(u) - 254.5 * jnp.log(1.0 - 0.25 * u)

    row_ids = jax.lax.broadcasted_iota(jnp.int32, (_BR, _B), 0) + step * _BR
    col_ids = jax.lax.broadcasted_iota(jnp.int32, (_BR, _B), 1)
    dmask = row_ids == col_ids
    valid = jnp.logical_not(dmask) & (u < 2.89)
    lwm = jnp.where(valid, lw, _NEG)

    pltpu.prng_seed(step + 1)
    b1 = pltpu.prng_random_bits((_BR, _B)).astype(jnp.uint32)
    b2 = pltpu.prng_random_bits((_BR, _B)).astype(jnp.uint32)
    k1 = lwm + _gumbel(b1)
    k2 = lwm + _gumbel(b2)

    rmax = jnp.max(k1, axis=1, keepdims=True)
    rpay = jnp.max(jnp.where(k1 >= rmax, s, _NEG), axis=1, keepdims=True)
    dvals = jnp.max(jnp.where(dmask, s, _NEG), axis=1, keepdims=True)
    part = jnp.sum(jnp.maximum(_MARGIN + rpay - dvals, 0.0))

    cmax = jnp.max(k2, axis=0, keepdims=True)
    cpay = jnp.max(jnp.where(k2 >= cmax, s, _NEG), axis=0, keepdims=True)

    @pl.when(step == 0)
    def _():
        colk[...] = cmax
        colp[...] = cpay
        diag[...] = jnp.zeros_like(diag)
        acc[0, 0] = 0.0

    @pl.when(step != 0)
    def _():
        take = cmax > colk[...]
        colp[...] = jnp.where(take, cpay, colp[...])
        colk[...] = jnp.maximum(cmax, colk[...])

    diag[...] += jnp.sum(jnp.where(dmask, s, 0.0), axis=0, keepdims=True)
    acc[0, 0] += part

    @pl.when(step == _NSTEPS - 1)
    def _():
        loss2 = jnp.sum(jnp.maximum(_MARGIN + colp[...] - diag[...], 0.0))
        out_ref[...] = jnp.broadcast_to(acc[0, 0] + loss2, (1, 1))


def kernel(sim_mat):
    out = pl.pallas_call(
        _body,
        grid=(_NSTEPS,),
        in_specs=[pl.BlockSpec((_BR, _B), lambda i: (i, 0))],
        out_specs=pl.BlockSpec((1, 1), lambda i: (0, 0)),
        out_shape=jax.ShapeDtypeStruct((1, 1), jnp.float32),
        scratch_shapes=[
            pltpu.VMEM((1, _B), jnp.float32),
            pltpu.VMEM((1, _B), jnp.float32),
            pltpu.VMEM((1, _B), jnp.float32),
            pltpu.SMEM((1, 1), jnp.float32),
        ],
        compiler_params=pltpu.CompilerParams(
            dimension_semantics=("arbitrary",),
        ),
    )(sim_mat)
    return out[0, 0]
